# baseline (speedup 1.0000x reference)
import math

import jax
import jax.numpy as jnp
import numpy as np
from jax.experimental import pallas as pl
from jax.experimental.pallas import tpu as pltpu

BLOCK_SIZE = 32
LOCAL_BLOCKS = 2
SINK_NUM = 4
WINDOW = LOCAL_BLOCKS * BLOCK_SIZE
S = 2048
DIM = 768
N_HEADS = 12
N_KV_HEADS = 4
N_REP = N_HEADS // N_KV_HEADS
HEAD_DIM = 64
KV_DIM = N_KV_HEADS * HEAD_DIM
Q_DIM = N_HEADS * HEAD_DIM

TQ = 128
TK = TQ + WINDOW
TSINK = 32
TC = TSINK + TK
NEG = -1e30
SCALE = 1.0 / math.sqrt(HEAD_DIM)


def _rot_pairs(t):
    r1 = jnp.roll(t, 1, axis=1)
    rm = jnp.roll(t, -1, axis=1)
    lane = jax.lax.broadcasted_iota(jnp.int32, t.shape, 1)
    return jnp.where(lane % 2 == 0, -rm, r1)


def _fused_kernel(x_ref, wq_ref, wk_ref, wv_ref, wo_ref, ci_ref, si_ref,
                  bias_ref, o_ref, kscr, vscr):
    i = pl.program_id(0)
    q0 = pl.multiple_of(i * TQ, TQ)
    ks = pl.multiple_of(jnp.maximum(q0 - WINDOW, 0), WINDOW)

    xt = x_ref[...]
    q = jnp.dot(xt, wq_ref[...], preferred_element_type=jnp.float32)
    k = jnp.dot(xt, wk_ref[...], preferred_element_type=jnp.float32)
    v = jnp.dot(xt, wv_ref[...], preferred_element_type=jnp.float32)

    ci = ci_ref[...]
    si = si_ref[...]
    ci12 = jnp.concatenate([ci] * N_HEADS, axis=1)
    si12 = jnp.concatenate([si] * N_HEADS, axis=1)
    ci4 = jnp.concatenate([ci] * N_KV_HEADS, axis=1)
    si4 = jnp.concatenate([si] * N_KV_HEADS, axis=1)

    qr = (q * ci12 + _rot_pairs(q) * si12) * SCALE
    kr = k * ci4 + _rot_pairs(k) * si4
    kscr[pl.ds(q0, TQ), :] = kr
    vscr[pl.ds(q0, TQ), :] = v

    @pl.when(i == 0)
    def _():
        kscr[pl.ds(TQ, WINDOW), :] = jnp.zeros((WINDOW, KV_DIM), jnp.float32)
        vscr[pl.ds(TQ, WINDOW), :] = jnp.zeros((WINDOW, KV_DIM), jnp.float32)

    bias = bias_ref[0]

    kcat, vcat = [], []
    for g in range(N_KV_HEADS):
        c = slice(g * HEAD_DIM, (g + 1) * HEAD_DIM)
        kcat.append(jnp.concatenate([kscr[0:TSINK, c], kscr[pl.ds(ks, TK), c]],
                                    axis=0))
        vcat.append(jnp.concatenate([vscr[0:TSINK, c], vscr[pl.ds(ks, TK), c]],
                                    axis=0))

    outs = []
    for h in range(N_HEADS):
        g = h // N_REP
        c = slice(h * HEAD_DIM, (h + 1) * HEAD_DIM)
        s = jax.lax.dot_general(qr[:, c], kcat[g], (((1,), (1,)), ((), ())),
                                preferred_element_type=jnp.float32)
        p = jnp.exp(s + bias)
        pv = jnp.dot(p, vcat[g], preferred_element_type=jnp.float32)
        denom = jnp.sum(p, axis=1, keepdims=True)
        outs.append(pv / denom)

    attn = jnp.concatenate(outs, axis=1)
    o_ref[...] = jnp.dot(attn, wo_ref[...], preferred_element_type=jnp.float32)


def _mask_bias():
    r = np.arange(TQ)[:, None]
    cs_ = np.arange(TSINK)[None, :]
    cw = np.arange(TK)[None, :]
    sink0 = np.zeros((TQ, TSINK), bool)
    win0 = (cw <= r) & ((cw >= r - WINDOW) | (cw < SINK_NUM))
    sink1 = np.broadcast_to(cs_ < SINK_NUM, (TQ, TSINK))
    win1 = (cw - WINDOW <= r) & (cw >= r)
    m = np.stack([np.concatenate([sink0, win0], axis=1),
                  np.concatenate([sink1, win1], axis=1)])
    return jnp.asarray(np.where(m, 0.0, NEG), dtype=jnp.float32)


def kernel(x, start_pos, freqs_cos, freqs_sin, wq, wk, wv, wo):
    del start_pos
    x2 = x[0]

    ci = jnp.repeat(freqs_cos, 2, axis=1)
    si = jnp.repeat(freqs_sin, 2, axis=1)
    bias = _mask_bias()

    nrow = S // TQ
    out = pl.pallas_call(
        _fused_kernel,
        grid=(nrow,),
        in_specs=[
            pl.BlockSpec((TQ, DIM), lambda r: (r, 0)),
            pl.BlockSpec((DIM, Q_DIM), lambda r: (0, 0)),
            pl.BlockSpec((DIM, KV_DIM), lambda r: (0, 0)),
            pl.BlockSpec((DIM, KV_DIM), lambda r: (0, 0)),
            pl.BlockSpec((DIM, DIM), lambda r: (0, 0)),
            pl.BlockSpec((TQ, HEAD_DIM), lambda r: (r, 0)),
            pl.BlockSpec((TQ, HEAD_DIM), lambda r: (r, 0)),
            pl.BlockSpec((1, TQ, TC), lambda r: (jnp.minimum(r, 1), 0, 0)),
        ],
        out_specs=pl.BlockSpec((TQ, DIM), lambda r: (r, 0)),
        out_shape=jax.ShapeDtypeStruct((S, DIM), jnp.float32),
        scratch_shapes=[
            pltpu.VMEM((S, KV_DIM), jnp.float32),
            pltpu.VMEM((S, KV_DIM), jnp.float32),
        ],
        compiler_params=pltpu.CompilerParams(
            dimension_semantics=("arbitrary",),
        ),
    )(x2, wq, wk, wv, wo, ci, si, bias)

    return out[None, :, :]


# speedup vs baseline: 1.6254x; 1.6254x over previous
import math

import jax
import jax.numpy as jnp
import numpy as np
from jax.experimental import pallas as pl
from jax.experimental.pallas import tpu as pltpu

BLOCK_SIZE = 32
LOCAL_BLOCKS = 2
SINK_NUM = 4
WINDOW = LOCAL_BLOCKS * BLOCK_SIZE
S = 2048
DIM = 768
N_HEADS = 12
N_KV_HEADS = 4
N_REP = N_HEADS // N_KV_HEADS
HEAD_DIM = 64
KV_DIM = N_KV_HEADS * HEAD_DIM
Q_DIM = N_HEADS * HEAD_DIM

TQ = 512
TK = TQ + WINDOW
TSINK = 32
TC = TSINK + TK
NEG = -1e30
SCALE = 1.0 / math.sqrt(HEAD_DIM)


def _rot_pairs(t):
    r1 = jnp.roll(t, 1, axis=1)
    rm = jnp.roll(t, -1, axis=1)
    lane = jax.lax.broadcasted_iota(jnp.int32, t.shape, 1)
    return jnp.where(lane % 2 == 0, -rm, r1)


def _fused_kernel(x_ref, wq_ref, wk_ref, wv_ref, wo_ref, ci_ref, si_ref,
                  bias_ref, o_ref, kscr, vscr):
    i = pl.program_id(0)
    q0 = pl.multiple_of(i * TQ, TQ)
    ks = pl.multiple_of(jnp.maximum(q0 - WINDOW, 0), WINDOW)

    xt = x_ref[...]
    q = jnp.dot(xt, wq_ref[...], preferred_element_type=jnp.float32)
    k = jnp.dot(xt, wk_ref[...], preferred_element_type=jnp.float32)
    v = jnp.dot(xt, wv_ref[...], preferred_element_type=jnp.float32)

    ci = ci_ref[...]
    si = si_ref[...]
    ci12 = jnp.concatenate([ci] * N_HEADS, axis=1)
    si12 = jnp.concatenate([si] * N_HEADS, axis=1)
    ci4 = jnp.concatenate([ci] * N_KV_HEADS, axis=1)
    si4 = jnp.concatenate([si] * N_KV_HEADS, axis=1)

    qr = (q * ci12 + _rot_pairs(q) * si12) * SCALE
    kr = k * ci4 + _rot_pairs(k) * si4
    kscr[pl.ds(q0, TQ), :] = kr
    vscr[pl.ds(q0, TQ), :] = v

    @pl.when(i == 0)
    def _():
        kscr[pl.ds(TQ, WINDOW), :] = jnp.zeros((WINDOW, KV_DIM), jnp.float32)
        vscr[pl.ds(TQ, WINDOW), :] = jnp.zeros((WINDOW, KV_DIM), jnp.float32)

    bias = bias_ref[0]

    kcat, vcat = [], []
    for g in range(N_KV_HEADS):
        c = slice(g * HEAD_DIM, (g + 1) * HEAD_DIM)
        kcat.append(jnp.concatenate([kscr[0:TSINK, c], kscr[pl.ds(ks, TK), c]],
                                    axis=0))
        vcat.append(jnp.concatenate([vscr[0:TSINK, c], vscr[pl.ds(ks, TK), c]],
                                    axis=0))

    outs = []
    for h in range(N_HEADS):
        g = h // N_REP
        c = slice(h * HEAD_DIM, (h + 1) * HEAD_DIM)
        s = jax.lax.dot_general(qr[:, c], kcat[g], (((1,), (1,)), ((), ())),
                                preferred_element_type=jnp.float32)
        p = jnp.exp(s + bias)
        pv = jnp.dot(p, vcat[g], preferred_element_type=jnp.float32)
        denom = jnp.sum(p, axis=1, keepdims=True)
        outs.append(pv / denom)

    attn = jnp.concatenate(outs, axis=1)
    o_ref[...] = jnp.dot(attn, wo_ref[...], preferred_element_type=jnp.float32)


def _mask_bias():
    r = np.arange(TQ)[:, None]
    cs_ = np.arange(TSINK)[None, :]
    cw = np.arange(TK)[None, :]
    sink0 = np.zeros((TQ, TSINK), bool)
    win0 = (cw <= r) & ((cw >= r - WINDOW) | (cw < SINK_NUM))
    sink1 = np.broadcast_to(cs_ < SINK_NUM, (TQ, TSINK))
    win1 = (cw - WINDOW <= r) & (cw >= r)
    m = np.stack([np.concatenate([sink0, win0], axis=1),
                  np.concatenate([sink1, win1], axis=1)])
    return jnp.asarray(np.where(m, 0.0, NEG), dtype=jnp.float32)


def kernel(x, start_pos, freqs_cos, freqs_sin, wq, wk, wv, wo):
    del start_pos
    x2 = x[0]

    ci = jnp.repeat(freqs_cos, 2, axis=1)
    si = jnp.repeat(freqs_sin, 2, axis=1)
    bias = _mask_bias()

    nrow = S // TQ
    out = pl.pallas_call(
        _fused_kernel,
        grid=(nrow,),
        in_specs=[
            pl.BlockSpec((TQ, DIM), lambda r: (r, 0)),
            pl.BlockSpec((DIM, Q_DIM), lambda r: (0, 0)),
            pl.BlockSpec((DIM, KV_DIM), lambda r: (0, 0)),
            pl.BlockSpec((DIM, KV_DIM), lambda r: (0, 0)),
            pl.BlockSpec((DIM, DIM), lambda r: (0, 0)),
            pl.BlockSpec((TQ, HEAD_DIM), lambda r: (r, 0)),
            pl.BlockSpec((TQ, HEAD_DIM), lambda r: (r, 0)),
            pl.BlockSpec((1, TQ, TC), lambda r: (jnp.minimum(r, 1), 0, 0)),
        ],
        out_specs=pl.BlockSpec((TQ, DIM), lambda r: (r, 0)),
        out_shape=jax.ShapeDtypeStruct((S, DIM), jnp.float32),
        scratch_shapes=[
            pltpu.VMEM((S, KV_DIM), jnp.float32),
            pltpu.VMEM((S, KV_DIM), jnp.float32),
        ],
        compiler_params=pltpu.CompilerParams(
            dimension_semantics=("arbitrary",),
        ),
    )(x2, wq, wk, wv, wo, ci, si, bias)

    return out[None, :, :]
